# baseline (speedup 1.0000x reference)
import functools

import jax
import jax.numpy as jnp
from jax.experimental import pallas as pl
from jax.experimental.pallas import tpu as pltpu

E = 8
TOP_K = 2


def _mm_nt_kernel(a_ref, b_ref, o_ref):
    a = a_ref[...].astype(jnp.bfloat16)
    b = b_ref[...].astype(jnp.bfloat16)
    o_ref[...] = jax.lax.dot_general(
        a, b, (((1,), (1,)), ((), ())), preferred_element_type=jnp.float32
    )


def _mm_nt(a, b, bm=256, bn=512):
    M, K = a.shape
    N, _ = b.shape
    bm = min(bm, M)
    bn = min(bn, N)
    grid = (M // bm, N // bn)
    return pl.pallas_call(
        _mm_nt_kernel,
        grid=grid,
        in_specs=[
            pl.BlockSpec((bm, K), lambda i, j: (i, 0)),
            pl.BlockSpec((bn, K), lambda i, j: (j, 0)),
        ],
        out_specs=pl.BlockSpec((bm, bn), lambda i, j: (i, j)),
        out_shape=jax.ShapeDtypeStruct((M, N), jnp.float32),
        compiler_params=pltpu.CompilerParams(
            dimension_semantics=("parallel", "parallel"),
        ),
    )(a, b)


def _glu_kernel(x_ref, wg_ref, wu_ref, o_ref):
    x = x_ref[...].astype(jnp.bfloat16)
    wg = wg_ref[...].astype(jnp.bfloat16)
    wu = wu_ref[...].astype(jnp.bfloat16)
    g = jax.lax.dot_general(
        x, wg, (((1,), (1,)), ((), ())), preferred_element_type=jnp.float32
    )
    u = jax.lax.dot_general(
        x, wu, (((1,), (1,)), ((), ())), preferred_element_type=jnp.float32
    )
    o_ref[...] = (g * jax.nn.sigmoid(g)) * u


def _glu(x, wg, wu, bm=256, bn=512):
    M, K = x.shape
    N, _ = wg.shape
    bn = min(bn, N)
    grid = (M // bm, N // bn)
    return pl.pallas_call(
        _glu_kernel,
        grid=grid,
        in_specs=[
            pl.BlockSpec((bm, K), lambda i, j: (i, 0)),
            pl.BlockSpec((bn, K), lambda i, j: (j, 0)),
            pl.BlockSpec((bn, K), lambda i, j: (j, 0)),
        ],
        out_specs=pl.BlockSpec((bm, bn), lambda i, j: (i, j)),
        out_shape=jax.ShapeDtypeStruct((M, N), jnp.float32),
        compiler_params=pltpu.CompilerParams(
            dimension_semantics=("parallel", "parallel"),
        ),
    )(x, wg, wu)


def kernel(hidden_states, router_weight, gate_up_proj, down_proj,
           shared_gate_w, shared_up_w, shared_down_w, shared_expert_gate_w):
    b, s, h = hidden_states.shape
    x = hidden_states.reshape(-1, h)
    N = x.shape[0]

    shared_act = _glu(x, shared_gate_w, shared_up_w)
    shared_expert_output = _mm_nt(shared_act, shared_down_w)

    router_logits = _mm_nt(x, router_weight, bm=N, bn=E)
    router_probs_full = jax.nn.softmax(router_logits, axis=-1)
    router_top_value, router_indices = jax.lax.top_k(router_probs_full, TOP_K)
    router_top_value = router_top_value / router_top_value.sum(axis=-1, keepdims=True)
    routing_weights = router_top_value

    final_hidden_states = jnp.zeros_like(x)
    for e in range(E):
        act = _glu(x, gate_up_proj[e, :1024], gate_up_proj[e, 1024:])
        y = _mm_nt(act, down_proj[e])
        w_e = jnp.sum(routing_weights * (router_indices == e), axis=-1)
        final_hidden_states = final_hidden_states + y * w_e[:, None]

    shared_gate_logit = x @ shared_expert_gate_w.T
    shared_gated = jax.nn.sigmoid(shared_gate_logit) * shared_expert_output
    expert_output = final_hidden_states + shared_gated

    expert_mask = jax.nn.one_hot(router_indices, E, dtype=jnp.float32)
    tokens_per_expert = expert_mask.sum(axis=(0, 1))
    fraction_tokens = tokens_per_expert / (N * TOP_K)
    router_probs_summed = jax.nn.softmax(router_logits, axis=-1).sum(axis=0)
    fraction_probs = router_probs_summed.sum() / N
    aux_loss = E * jnp.sum(fraction_tokens * fraction_probs)

    return (expert_output.reshape(b, s, h), aux_loss)


# speedup vs baseline: 2.2137x; 2.2137x over previous
import functools

import jax
import jax.numpy as jnp
from jax.experimental import pallas as pl
from jax.experimental.pallas import tpu as pltpu

E = 8
TOP_K = 2


def _dot_nt(a, b):
    return jax.lax.dot_general(
        a, b, (((1,), (1,)), ((), ())), preferred_element_type=jnp.float32
    )


def _router_kernel(x_ref, rw_ref, o_ref):
    o_ref[...] = _dot_nt(x_ref[...], rw_ref[...])


def _shared_kernel(nj, x_ref, sg_ref, su_ref, sd_ref, sgw_ref, o_ref, logit_ref):
    j = pl.program_id(0)
    x = x_ref[...]

    @pl.when(j == 0)
    def _():
        prod = x.astype(jnp.float32) * sgw_ref[...]
        logit_ref[...] = jnp.sum(prod, axis=1, keepdims=True)
        o_ref[...] = jnp.zeros_like(o_ref)

    g = _dot_nt(x, sg_ref[...])
    u = _dot_nt(x, su_ref[...])
    act = ((g * jax.nn.sigmoid(g)) * u).astype(jnp.bfloat16)
    o_ref[...] += _dot_nt(act, sd_ref[...])

    @pl.when(j == nj - 1)
    def _():
        o_ref[...] = o_ref[...] * jax.nn.sigmoid(logit_ref[...])


def _experts_kernel(x_ref, gw_ref, uw_ref, dw_ref, w_ref, sh_ref, o_ref):
    e = pl.program_id(1)
    x = x_ref[...]
    g = _dot_nt(x, gw_ref[0])
    u = _dot_nt(x, uw_ref[0])
    act = (g * jax.nn.sigmoid(g)) * u
    act = (act * w_ref[0]).astype(jnp.bfloat16)
    ys = _dot_nt(act, dw_ref[0])

    @pl.when(e == 0)
    def _():
        o_ref[...] = sh_ref[...] + ys

    @pl.when(e > 0)
    def _():
        o_ref[...] += ys


def kernel(hidden_states, router_weight, gate_up_proj, down_proj,
           shared_gate_w, shared_up_w, shared_down_w, shared_expert_gate_w):
    b, s, h = hidden_states.shape
    x = hidden_states.reshape(-1, h)
    N = x.shape[0]
    I = down_proj.shape[2]
    SI = shared_gate_w.shape[0]

    xb = x.astype(jnp.bfloat16)
    gw = gate_up_proj[:, :I].astype(jnp.bfloat16)
    uw = gate_up_proj[:, I:].astype(jnp.bfloat16)
    dw = down_proj.astype(jnp.bfloat16)
    sg = shared_gate_w.astype(jnp.bfloat16)
    su = shared_up_w.astype(jnp.bfloat16)
    sd = shared_down_w.astype(jnp.bfloat16)
    sgw = shared_expert_gate_w.astype(jnp.float32)

    router_logits = pl.pallas_call(
        _router_kernel,
        out_shape=jax.ShapeDtypeStruct((N, E), jnp.float32),
    )(xb, router_weight.astype(jnp.bfloat16))

    router_probs_full = jax.nn.softmax(router_logits, axis=-1)
    router_top_value, router_indices = jax.lax.top_k(router_probs_full, TOP_K)
    router_top_value = router_top_value / router_top_value.sum(axis=-1, keepdims=True)
    w_table = jnp.zeros((N, E), jnp.float32)
    w_table = jnp.sum(
        router_top_value[..., None] * jax.nn.one_hot(router_indices, E, dtype=jnp.float32),
        axis=1,
    )
    w_exp = w_table.T.reshape(E, N, 1)

    BS = 512
    NJ = SI // BS
    shared_gated = pl.pallas_call(
        functools.partial(_shared_kernel, NJ),
        grid=(NJ,),
        in_specs=[
            pl.BlockSpec((N, h), lambda j: (0, 0)),
            pl.BlockSpec((BS, h), lambda j: (j, 0)),
            pl.BlockSpec((BS, h), lambda j: (j, 0)),
            pl.BlockSpec((h, BS), lambda j: (0, j)),
            pl.BlockSpec((1, h), lambda j: (0, 0)),
        ],
        out_specs=pl.BlockSpec((N, h), lambda j: (0, 0)),
        out_shape=jax.ShapeDtypeStruct((N, h), jnp.float32),
        scratch_shapes=[pltpu.VMEM((N, 1), jnp.float32)],
        compiler_params=pltpu.CompilerParams(
            dimension_semantics=("arbitrary",),
        ),
    )(xb, sg, su, sd, sgw)

    BM = 512
    R = N // BM
    expert_output = pl.pallas_call(
        _experts_kernel,
        grid=(R, E),
        in_specs=[
            pl.BlockSpec((BM, h), lambda i, e: (i, 0)),
            pl.BlockSpec((1, I, h), lambda i, e: (e, 0, 0)),
            pl.BlockSpec((1, I, h), lambda i, e: (e, 0, 0)),
            pl.BlockSpec((1, h, I), lambda i, e: (e, 0, 0)),
            pl.BlockSpec((1, BM, 1), lambda i, e: (e, i, 0)),
            pl.BlockSpec((BM, h), lambda i, e: (i, 0)),
        ],
        out_specs=pl.BlockSpec((BM, h), lambda i, e: (i, 0)),
        out_shape=jax.ShapeDtypeStruct((N, h), jnp.float32),
        compiler_params=pltpu.CompilerParams(
            dimension_semantics=("parallel", "arbitrary"),
        ),
    )(xb, gw, uw, dw, w_exp, shared_gated)

    expert_mask = jax.nn.one_hot(router_indices, E, dtype=jnp.float32)
    tokens_per_expert = expert_mask.sum(axis=(0, 1))
    fraction_tokens = tokens_per_expert / (N * TOP_K)
    router_probs_summed = jax.nn.softmax(router_logits, axis=-1).sum(axis=0)
    fraction_probs = router_probs_summed.sum() / N
    aux_loss = E * jnp.sum(fraction_tokens * fraction_probs)

    return (expert_output.reshape(b, s, h), aux_loss)
